# baseline (speedup 1.0000x reference)
import functools

import jax
import jax.numpy as jnp
from jax import lax
from jax.experimental import pallas as pl
from jax.experimental.pallas import tpu as pltpu
from jax.experimental.pallas import tpu_sc as plsc

_N = 10000
_E = 320000
_H = 128
_NW = 32
_NPW = 320
_NPAD = _NW * _NPW
_CK = 64
_SG = 32
_EPG = _CK * _SG
_EPAD = ((_E + _EPG - 1) // _EPG) * _EPG
_HEAD_CK = 128
_HEAD_CH = (_N + _HEAD_CK - 1) // _HEAD_CK
_HEAD_PAD = _HEAD_CH * _HEAD_CK


def _mm_kernel(x_ref, w_ref, b_ref, o_ref, *, relu):
    y = jnp.dot(x_ref[...], w_ref[...], preferred_element_type=jnp.float32)
    y = y + b_ref[...]
    if relu:
        y = jnp.maximum(y, 0.0)
    o_ref[...] = y


def _mm(x, w, b, relu, br):
    r_rows, k = x.shape
    h = w.shape[1]
    return pl.pallas_call(
        functools.partial(_mm_kernel, relu=relu),
        grid=(r_rows // br,),
        in_specs=[
            pl.BlockSpec((br, k), lambda i: (i, 0)),
            pl.BlockSpec((k, h), lambda i: (0, 0)),
            pl.BlockSpec((1, h), lambda i: (0, 0)),
        ],
        out_specs=pl.BlockSpec((br, h), lambda i: (i, 0)),
        out_shape=jax.ShapeDtypeStruct((r_rows, h), jnp.float32),
    )(x, w, b.reshape(1, h))


def _layer_tc_kernel(h_ref, g_ref, head_ref, w_ref, b_ref, h2_ref, a_ref, bt_ref):
    h_new = h_ref[...] * g_ref[...]
    h2_ref[...] = h_new
    a_ref[...] = (jnp.dot(h_new, w_ref[...], preferred_element_type=jnp.float32)
                  + b_ref[...])
    bt_ref[...] = -jnp.dot(head_ref[...], w_ref[...],
                           preferred_element_type=jnp.float32)


def _layer_tc(h_atom, aggr, head, w, b, br=2000):
    n, h = h_atom.shape
    spec = pl.BlockSpec((br, h), lambda i: (i, 0))
    return pl.pallas_call(
        _layer_tc_kernel,
        grid=(n // br,),
        in_specs=[spec, spec, spec,
                  pl.BlockSpec((h, h), lambda i: (0, 0)),
                  pl.BlockSpec((1, h), lambda i: (0, 0))],
        out_specs=[spec, spec, spec],
        out_shape=[jax.ShapeDtypeStruct((n, h), jnp.float32)] * 3,
    )(h_atom, aggr, head, w, b.reshape(1, h))


def _final_tc_kernel(h_ref, g_ref, xp_ref, w1_ref, w2_ref, b_ref, o_ref):
    h = h_ref[...]
    hf = h * (h * g_ref[...])
    o_ref[...] = (jnp.dot(hf, w1_ref[...], preferred_element_type=jnp.float32)
                  + jnp.dot(xp_ref[...], w2_ref[...],
                            preferred_element_type=jnp.float32)
                  + b_ref[...])


def _final_tc(h_atom, aggr, x_proj, wlin, blin, br=2000):
    n, h = h_atom.shape
    out = wlin.shape[1]
    spec = pl.BlockSpec((br, h), lambda i: (i, 0))
    return pl.pallas_call(
        _final_tc_kernel,
        grid=(n // br,),
        in_specs=[spec, spec, spec,
                  pl.BlockSpec((h, out), lambda i: (0, 0)),
                  pl.BlockSpec((h, out), lambda i: (0, 0)),
                  pl.BlockSpec((1, out), lambda i: (0, 0))],
        out_specs=pl.BlockSpec((br, out), lambda i: (i, 0)),
        out_shape=jax.ShapeDtypeStruct((n, out), jnp.float32),
    )(h_atom, aggr, x_proj, wlin[:h], wlin[h:], blin.reshape(1, out))


_MESH_CACHE = []


def _mesh():
    if not _MESH_CACHE:
        _MESH_CACHE.append(plsc.VectorSubcoreMesh(
            core_axis_name="c", subcore_axis_name="s",
            num_cores=2, num_subcores=16))
    return _MESH_CACHE[0]


def _sweep_body(gather, a_hbm, bt_hbm, src_hbm, dst_hbm, bounds_hbm, off_hbm,
                out_hbm, isrc, igat, bounds_vm, off_vm, acc_s, acc_m,
                ba0, bb0, ba1, bb1, sa0, sb0, sa1, sb1):
    w = lax.axis_index("s") * 2 + lax.axis_index("c")
    nbase = w * _NPW
    zero16 = jnp.zeros((16,), jnp.float32)

    def zero_row(i, _):
        for j in range(8):
            acc_s[i, 16 * j:16 * (j + 1)] = zero16
            acc_m[i, 16 * j:16 * (j + 1)] = zero16
        return 0

    lax.fori_loop(0, _NPW, zero_row, 0)

    pltpu.sync_copy(bounds_hbm, bounds_vm)
    pltpu.sync_copy(off_hbm.at[pl.ds(nbase, _NPW + 16)], off_vm)
    bv = bounds_vm[pl.ds(w, 16)]
    e0 = bv[0]
    e1 = bv[1]
    c_lo = e0 // _CK
    c_hi = (e1 + _CK - 1) // _CK

    def issue(ci, buf_a, buf_b, sem_a, sem_b):
        k = (ci - (ci // _SG) * _SG) * _CK
        ia = isrc.at[pl.ds(k, _CK)]
        ig = igat.at[pl.ds(k, _CK)]
        if gather:
            pltpu.async_copy(a_hbm.at[ia], buf_a, sem_a)
            pltpu.async_copy(bt_hbm.at[ig], buf_b, sem_b)
        else:
            pltpu.async_copy(a_hbm.at[ig], buf_a, sem_a)

    def wait_bufs(buf_a, buf_b, sem_a, sem_b):
        pltpu.make_async_copy(a_hbm.at[pl.ds(0, _CK)], buf_a, sem_a).wait()
        if gather:
            pltpu.make_async_copy(bt_hbm.at[pl.ds(0, _CK)], buf_b, sem_b).wait()

    def process(ci, buf_a, buf_b):
        start = ci * _CK
        k = (ci - (ci // _SG) * _SG) * _CK
        lo = jnp.maximum(e0 - start, 0)
        hi = jnp.minimum(e1 - start, _CK)

        @pl.when(lo < hi)
        def _():
            n_lo = isrc[pl.ds(k + lo, 16)][0]
            n_hi = isrc[pl.ds(k + hi - 1, 16)][0]

            def node(n, _):
                r = n - nbase
                ofs = off_vm[pl.ds(r, 16)]
                el = jnp.maximum(ofs[0] - start, lo)
                eh = jnp.minimum(ofs[1] - start, hi)

                def edge(e, regs):
                    new = []
                    for j in range(8):
                        sl = pl.ds(16 * j, 16)
                        v = buf_a[e, sl]
                        if gather:
                            v = jnp.maximum(v + buf_b[e, sl], 0.0)
                        new.append((regs[j] + v,
                                    jnp.maximum(regs[8 + j], v)))
                    return (tuple(p[0] for p in new)
                            + tuple(p[1] for p in new))

                regs = lax.fori_loop(el, eh, edge, (zero16,) * 16)
                for j in range(8):
                    sl = pl.ds(16 * j, 16)
                    acc_s[r, sl] = acc_s[r, sl] + regs[j]
                    acc_m[r, sl] = jnp.maximum(acc_m[r, sl], regs[8 + j])
                return 0

            lax.fori_loop(n_lo, n_hi + 1, node, 0)

    def sgroup(sg, _):
        cl = jnp.maximum(c_lo, sg * _SG)
        ch = jnp.minimum(c_hi, (sg + 1) * _SG)

        @pl.when(cl < ch)
        def _():
            base = sg * _EPG
            pltpu.sync_copy(src_hbm.at[pl.ds(base, _EPG)],
                            isrc.at[pl.ds(0, _EPG)])
            pltpu.sync_copy(dst_hbm.at[pl.ds(base, _EPG)], igat)
            issue(cl, ba0, bb0, sa0, sb0)

            def pair(p, _):
                ci0 = cl + 2 * p
                ci1 = ci0 + 1

                @pl.when(ci1 < ch)
                def _():
                    issue(ci1, ba1, bb1, sa1, sb1)

                wait_bufs(ba0, bb0, sa0, sb0)
                process(ci0, ba0, bb0)

                @pl.when(ci1 < ch)
                def _():
                    @pl.when(ci1 + 1 < ch)
                    def _():
                        issue(ci1 + 1, ba0, bb0, sa0, sb0)

                    wait_bufs(ba1, bb1, sa1, sb1)
                    process(ci1, ba1, bb1)

                return 0

            lax.fori_loop(0, (ch - cl + 1) // 2, pair, 0)

        return 0

    lax.fori_loop(c_lo // _SG, (c_hi + _SG - 1) // _SG, sgroup, 0)

    def finish(i, _):
        for j in range(8):
            sl = pl.ds(16 * j, 16)
            acc_s[i, sl] = acc_s[i, sl] * acc_m[i, sl]
        return 0

    lax.fori_loop(0, _NPW, finish, 0)
    pltpu.sync_copy(acc_s, out_hbm.at[pl.ds(nbase, _NPW)])


def _make_sweep(gather):
    scratch = [
        pltpu.VMEM((_EPG + 16,), jnp.int32),
        pltpu.VMEM((_EPG,), jnp.int32),
        pltpu.VMEM((48,), jnp.int32),
        pltpu.VMEM((_NPW + 16,), jnp.int32),
        pltpu.VMEM((_NPW, _H), jnp.float32),
        pltpu.VMEM((_NPW, _H), jnp.float32),
        pltpu.VMEM((_CK, _H), jnp.float32),
        pltpu.VMEM((_CK, _H), jnp.float32),
        pltpu.VMEM((_CK, _H), jnp.float32),
        pltpu.VMEM((_CK, _H), jnp.float32),
        pltpu.SemaphoreType.DMA,
        pltpu.SemaphoreType.DMA,
        pltpu.SemaphoreType.DMA,
        pltpu.SemaphoreType.DMA,
    ]
    return pl.kernel(
        functools.partial(_sweep_body, gather),
        mesh=_mesh(),
        out_type=jax.ShapeDtypeStruct((_NPAD, _H), jnp.float32),
        scratch_types=scratch,
    )


def _head_body(a_hbm, bt_hbm, srch_hbm, dsth_hbm, out_hbm,
               idxa, idxb, buf_a, buf_b, sem0, sem1):
    w = lax.axis_index("s") * 2 + lax.axis_index("c")

    def chunk(ci, _):
        c = w + ci * _NW

        @pl.when(c < _HEAD_CH)
        def _():
            start = c * _HEAD_CK
            pltpu.sync_copy(srch_hbm.at[pl.ds(start, _HEAD_CK)], idxa)
            pltpu.sync_copy(dsth_hbm.at[pl.ds(start, _HEAD_CK)], idxb)
            cp_a = pltpu.async_copy(a_hbm.at[idxa], buf_a, sem0)
            cp_b = pltpu.async_copy(bt_hbm.at[idxb], buf_b, sem1)
            cp_a.wait()
            cp_b.wait()

            def rowf(e, _):
                for j in range(8):
                    sl = pl.ds(16 * j, 16)
                    buf_a[e, sl] = jnp.maximum(buf_a[e, sl] + buf_b[e, sl], 0.0)
                return 0

            lax.fori_loop(0, _HEAD_CK, rowf, 0)
            pltpu.sync_copy(buf_a, out_hbm.at[pl.ds(start, _HEAD_CK)])

        return 0

    lax.fori_loop(0, (_HEAD_CH + _NW - 1) // _NW, chunk, 0)


_head_kernel = None


def _make_head():
    global _head_kernel
    if _head_kernel is None:
        scratch = [
            pltpu.VMEM((_HEAD_CK,), jnp.int32),
            pltpu.VMEM((_HEAD_CK,), jnp.int32),
            pltpu.VMEM((_HEAD_CK, _H), jnp.float32),
            pltpu.VMEM((_HEAD_CK, _H), jnp.float32),
            pltpu.SemaphoreType.DMA,
            pltpu.SemaphoreType.DMA,
        ]
        _head_kernel = pl.kernel(
            _head_body,
            mesh=_mesh(),
            out_type=jax.ShapeDtypeStruct((_HEAD_PAD, _H), jnp.float32),
            scratch_types=scratch,
        )
    return _head_kernel


_sweep_gather = None
_sweep_linear = None


def _get_sweeps():
    global _sweep_gather, _sweep_linear
    if _sweep_gather is None:
        _sweep_gather = _make_sweep(True)
        _sweep_linear = _make_sweep(False)
    return _sweep_gather, _sweep_linear


def kernel(x, edge_index, edge_attr, Wa, ba, Wb, bb, Wseq, bseq, Wlin, blin):
    sweep_gather, sweep_linear = _get_sweeps()
    head_fn = _make_head()

    src = edge_index[0]
    dst = edge_index[1]
    perm = jnp.argsort(src).astype(jnp.int32)
    src_s = src[perm].astype(jnp.int32)
    dst_s = dst[perm].astype(jnp.int32)
    node_off = jnp.searchsorted(
        src_s, jnp.arange(_NPAD + 16, dtype=jnp.int32),
        method='sort').astype(jnp.int32)
    bounds = node_off[0:_NPAD + 1:_NPW]
    perm = jnp.pad(perm, (0, _EPAD - _E))
    src_s = jnp.pad(src_s, (0, _EPAD - _E))
    dst_s = jnp.pad(dst_s, (0, _EPAD - _E))
    bounds = jnp.pad(bounds, (0, 48 - _NW - 1), constant_values=_E)
    srch = jnp.pad(src[:_N], (0, _HEAD_PAD - _N)).astype(jnp.int32)
    dsth = jnp.pad(dst[:_N], (0, _HEAD_PAD - _N)).astype(jnp.int32)

    x_proj = _mm(x, Wa, ba, True, 2000)
    r0 = _mm(edge_attr, Wb, bb, True, 2000)
    head = r0[:_N]
    h_atom = x_proj

    for l in range(3):
        if l == 0:
            aggr = sweep_linear(r0, r0, src_s, perm, bounds, node_off)[:_N]
        else:
            aggr = sweep_gather(a_tab, bt_tab, src_s, dst_s, bounds,
                                node_off)[:_N]
        h_atom, a_tab, bt_tab = _layer_tc(h_atom, aggr, head, Wseq[l], bseq[l])
        head = head_fn(a_tab, bt_tab, srch, dsth)[:_N]
    aggr = sweep_gather(a_tab, bt_tab, src_s, dst_s, bounds, node_off)[:_N]
    return _final_tc(h_atom, aggr, x_proj, Wlin, blin)


# speedup vs baseline: 1.5154x; 1.5154x over previous
import functools

import jax
import jax.numpy as jnp
from jax import lax
from jax.experimental import pallas as pl
from jax.experimental.pallas import tpu as pltpu
from jax.experimental.pallas import tpu_sc as plsc

_N = 10000
_E = 320000
_H = 128
_NW = 32
_NPW = 320
_NPAD = _NW * _NPW
_CK = 64
_SG = 32
_EPG = _CK * _SG
_EPAD = ((_E + _EPG - 1) // _EPG) * _EPG
_HEAD_CK = 128
_HEAD_CH = (_N + _HEAD_CK - 1) // _HEAD_CK
_HEAD_PAD = _HEAD_CH * _HEAD_CK


def _mm_kernel(x_ref, w_ref, b_ref, o_ref, *, relu):
    y = jnp.dot(x_ref[...], w_ref[...], preferred_element_type=jnp.float32)
    y = y + b_ref[...]
    if relu:
        y = jnp.maximum(y, 0.0)
    o_ref[...] = y


def _mm(x, w, b, relu, br):
    r_rows, k = x.shape
    h = w.shape[1]
    return pl.pallas_call(
        functools.partial(_mm_kernel, relu=relu),
        grid=(r_rows // br,),
        in_specs=[
            pl.BlockSpec((br, k), lambda i: (i, 0)),
            pl.BlockSpec((k, h), lambda i: (0, 0)),
            pl.BlockSpec((1, h), lambda i: (0, 0)),
        ],
        out_specs=pl.BlockSpec((br, h), lambda i: (i, 0)),
        out_shape=jax.ShapeDtypeStruct((r_rows, h), jnp.float32),
    )(x, w, b.reshape(1, h))


def _layer_tc_kernel(h_ref, g_ref, head_ref, w_ref, b_ref, h2_ref, a_ref, bt_ref):
    h_new = h_ref[...] * g_ref[...]
    h2_ref[...] = h_new
    a_ref[...] = (jnp.dot(h_new, w_ref[...], preferred_element_type=jnp.float32)
                  + b_ref[...])
    bt_ref[...] = -jnp.dot(head_ref[...], w_ref[...],
                           preferred_element_type=jnp.float32)


def _layer_tc(h_atom, aggr, head, w, b, br=2000):
    n, h = h_atom.shape
    spec = pl.BlockSpec((br, h), lambda i: (i, 0))
    return pl.pallas_call(
        _layer_tc_kernel,
        grid=(n // br,),
        in_specs=[spec, spec, spec,
                  pl.BlockSpec((h, h), lambda i: (0, 0)),
                  pl.BlockSpec((1, h), lambda i: (0, 0))],
        out_specs=[spec, spec, spec],
        out_shape=[jax.ShapeDtypeStruct((n, h), jnp.float32)] * 3,
    )(h_atom, aggr, head, w, b.reshape(1, h))


def _final_tc_kernel(h_ref, g_ref, xp_ref, w1_ref, w2_ref, b_ref, o_ref):
    h = h_ref[...]
    hf = h * (h * g_ref[...])
    o_ref[...] = (jnp.dot(hf, w1_ref[...], preferred_element_type=jnp.float32)
                  + jnp.dot(xp_ref[...], w2_ref[...],
                            preferred_element_type=jnp.float32)
                  + b_ref[...])


def _final_tc(h_atom, aggr, x_proj, wlin, blin, br=2000):
    n, h = h_atom.shape
    out = wlin.shape[1]
    spec = pl.BlockSpec((br, h), lambda i: (i, 0))
    return pl.pallas_call(
        _final_tc_kernel,
        grid=(n // br,),
        in_specs=[spec, spec, spec,
                  pl.BlockSpec((h, out), lambda i: (0, 0)),
                  pl.BlockSpec((h, out), lambda i: (0, 0)),
                  pl.BlockSpec((1, out), lambda i: (0, 0))],
        out_specs=pl.BlockSpec((br, out), lambda i: (i, 0)),
        out_shape=jax.ShapeDtypeStruct((n, out), jnp.float32),
    )(h_atom, aggr, x_proj, wlin[:h], wlin[h:], blin.reshape(1, out))


_MESH_CACHE = []


def _mesh():
    if not _MESH_CACHE:
        _MESH_CACHE.append(plsc.VectorSubcoreMesh(
            core_axis_name="c", subcore_axis_name="s",
            num_cores=2, num_subcores=16))
    return _MESH_CACHE[0]


def _sweep_body(gather, a_hbm, bt_hbm, src_hbm, dst_hbm, bounds_hbm, off_hbm,
                out_hbm, isrc, igat, bounds_vm, off_vm, acc_s, acc_m,
                ba0, bb0, ba1, bb1, sa0, sb0, sa1, sb1):
    w = lax.axis_index("s") * 2 + lax.axis_index("c")
    nbase = w * _NPW
    zero16 = jnp.zeros((16,), jnp.float32)

    def zero_row(i, _):
        for j in range(8):
            acc_s[i, 16 * j:16 * (j + 1)] = zero16
            acc_m[i, 16 * j:16 * (j + 1)] = zero16
        return 0

    lax.fori_loop(0, _NPW, zero_row, 0)

    pltpu.sync_copy(bounds_hbm, bounds_vm)
    pltpu.sync_copy(off_hbm.at[pl.ds(nbase, _NPW + 16)], off_vm)
    bv = bounds_vm[pl.ds(w, 16)]
    e0 = bv[0]
    e1 = bv[1]
    c_lo = e0 // _CK
    c_hi = (e1 + _CK - 1) // _CK

    def issue(ci, buf_a, buf_b, sem_a, sem_b):
        k = (ci - (ci // _SG) * _SG) * _CK
        ia = isrc.at[pl.ds(k, _CK)]
        ig = igat.at[pl.ds(k, _CK)]
        if gather:
            pltpu.async_copy(a_hbm.at[ia], buf_a, sem_a)
            pltpu.async_copy(bt_hbm.at[ig], buf_b, sem_b)
        else:
            pltpu.async_copy(a_hbm.at[ig], buf_a, sem_a)

    def wait_bufs(buf_a, buf_b, sem_a, sem_b):
        pltpu.make_async_copy(a_hbm.at[pl.ds(0, _CK)], buf_a, sem_a).wait()
        if gather:
            pltpu.make_async_copy(bt_hbm.at[pl.ds(0, _CK)], buf_b, sem_b).wait()

    def process(ci, buf_a, buf_b):
        start = ci * _CK
        k = (ci - (ci // _SG) * _SG) * _CK
        lo = jnp.maximum(e0 - start, 0)
        hi = jnp.minimum(e1 - start, _CK)

        @pl.when(lo < hi)
        def _():
            n_lo = isrc[pl.ds(k + lo, 16)][0]
            n_hi = isrc[pl.ds(k + hi - 1, 16)][0]

            def node(n, _):
                r = n - nbase
                ofs = off_vm[pl.ds(r, 16)]
                el = jnp.maximum(ofs[0] - start, lo)
                eh = jnp.minimum(ofs[1] - start, hi)

                def edge(e, regs):
                    new = []
                    for j in range(8):
                        sl = pl.ds(16 * j, 16)
                        v = buf_a[e, sl]
                        if gather:
                            v = jnp.maximum(v + buf_b[e, sl], 0.0)
                        new.append((regs[j] + v,
                                    jnp.maximum(regs[8 + j], v)))
                    return (tuple(p[0] for p in new)
                            + tuple(p[1] for p in new))

                regs = lax.fori_loop(el, eh, edge, (zero16,) * 16)
                for j in range(8):
                    sl = pl.ds(16 * j, 16)
                    acc_s[r, sl] = acc_s[r, sl] + regs[j]
                    acc_m[r, sl] = jnp.maximum(acc_m[r, sl], regs[8 + j])
                return 0

            lax.fori_loop(n_lo, n_hi + 1, node, 0)

    def sgroup(sg, _):
        cl = jnp.maximum(c_lo, sg * _SG)
        ch = jnp.minimum(c_hi, (sg + 1) * _SG)

        @pl.when(cl < ch)
        def _():
            base = sg * _EPG
            pltpu.sync_copy(src_hbm.at[pl.ds(base, _EPG)],
                            isrc.at[pl.ds(0, _EPG)])
            pltpu.sync_copy(dst_hbm.at[pl.ds(base, _EPG)], igat)
            issue(cl, ba0, bb0, sa0, sb0)

            def pair(p, _):
                ci0 = cl + 2 * p
                ci1 = ci0 + 1

                @pl.when(ci1 < ch)
                def _():
                    issue(ci1, ba1, bb1, sa1, sb1)

                wait_bufs(ba0, bb0, sa0, sb0)
                process(ci0, ba0, bb0)

                @pl.when(ci1 < ch)
                def _():
                    @pl.when(ci1 + 1 < ch)
                    def _():
                        issue(ci1 + 1, ba0, bb0, sa0, sb0)

                    wait_bufs(ba1, bb1, sa1, sb1)
                    process(ci1, ba1, bb1)

                return 0

            lax.fori_loop(0, (ch - cl + 1) // 2, pair, 0)

        return 0

    lax.fori_loop(c_lo // _SG, (c_hi + _SG - 1) // _SG, sgroup, 0)

    def finish(i, _):
        for j in range(8):
            sl = pl.ds(16 * j, 16)
            acc_s[i, sl] = acc_s[i, sl] * acc_m[i, sl]
        return 0

    lax.fori_loop(0, _NPW, finish, 0)
    pltpu.sync_copy(acc_s, out_hbm.at[pl.ds(nbase, _NPW)])


def _make_sweep(gather):
    scratch = [
        pltpu.VMEM((_EPG + 16,), jnp.int32),
        pltpu.VMEM((_EPG,), jnp.int32),
        pltpu.VMEM((48,), jnp.int32),
        pltpu.VMEM((_NPW + 16,), jnp.int32),
        pltpu.VMEM((_NPW, _H), jnp.float32),
        pltpu.VMEM((_NPW, _H), jnp.float32),
        pltpu.VMEM((_CK, _H), jnp.float32),
        pltpu.VMEM((_CK, _H), jnp.float32),
        pltpu.VMEM((_CK, _H), jnp.float32),
        pltpu.VMEM((_CK, _H), jnp.float32),
        pltpu.SemaphoreType.DMA,
        pltpu.SemaphoreType.DMA,
        pltpu.SemaphoreType.DMA,
        pltpu.SemaphoreType.DMA,
    ]
    return pl.kernel(
        functools.partial(_sweep_body, gather),
        mesh=_mesh(),
        out_type=jax.ShapeDtypeStruct((_NPAD, _H), jnp.float32),
        scratch_types=scratch,
    )


def _head_body(a_hbm, bt_hbm, srch_hbm, dsth_hbm, out_hbm,
               idxa, idxb, buf_a, buf_b, sem0, sem1):
    w = lax.axis_index("s") * 2 + lax.axis_index("c")

    def chunk(ci, _):
        c = w + ci * _NW

        @pl.when(c < _HEAD_CH)
        def _():
            start = c * _HEAD_CK
            pltpu.sync_copy(srch_hbm.at[pl.ds(start, _HEAD_CK)], idxa)
            pltpu.sync_copy(dsth_hbm.at[pl.ds(start, _HEAD_CK)], idxb)
            cp_a = pltpu.async_copy(a_hbm.at[idxa], buf_a, sem0)
            cp_b = pltpu.async_copy(bt_hbm.at[idxb], buf_b, sem1)
            cp_a.wait()
            cp_b.wait()

            def rowf(e, _):
                for j in range(8):
                    sl = pl.ds(16 * j, 16)
                    buf_a[e, sl] = jnp.maximum(buf_a[e, sl] + buf_b[e, sl], 0.0)
                return 0

            lax.fori_loop(0, _HEAD_CK, rowf, 0)
            pltpu.sync_copy(buf_a, out_hbm.at[pl.ds(start, _HEAD_CK)])

        return 0

    lax.fori_loop(0, (_HEAD_CH + _NW - 1) // _NW, chunk, 0)


_head_kernel = None


def _make_head():
    global _head_kernel
    if _head_kernel is None:
        scratch = [
            pltpu.VMEM((_HEAD_CK,), jnp.int32),
            pltpu.VMEM((_HEAD_CK,), jnp.int32),
            pltpu.VMEM((_HEAD_CK, _H), jnp.float32),
            pltpu.VMEM((_HEAD_CK, _H), jnp.float32),
            pltpu.SemaphoreType.DMA,
            pltpu.SemaphoreType.DMA,
        ]
        _head_kernel = pl.kernel(
            _head_body,
            mesh=_mesh(),
            out_type=jax.ShapeDtypeStruct((_HEAD_PAD, _H), jnp.float32),
            scratch_types=scratch,
        )
    return _head_kernel


_sweep_gather = None
_sweep_linear = None


def _get_sweeps():
    global _sweep_gather, _sweep_linear
    if _sweep_gather is None:
        _sweep_gather = _make_sweep(True)
        _sweep_linear = _make_sweep(False)
    return _sweep_gather, _sweep_linear


def kernel(x, edge_index, edge_attr, Wa, ba, Wb, bb, Wseq, bseq, Wlin, blin):
    sweep_gather, sweep_linear = _get_sweeps()
    head_fn = _make_head()

    src = edge_index[0]
    dst = edge_index[1]
    perm = jnp.argsort(src).astype(jnp.int32)
    src_s = src[perm].astype(jnp.int32)
    dst_s = dst[perm].astype(jnp.int32)
    node_off = jnp.searchsorted(
        src_s, jnp.arange(_NPAD + 16, dtype=jnp.int32)).astype(jnp.int32)
    bounds = node_off[0:_NPAD + 1:_NPW]
    perm = jnp.pad(perm, (0, _EPAD - _E))
    src_s = jnp.pad(src_s, (0, _EPAD - _E))
    dst_s = jnp.pad(dst_s, (0, _EPAD - _E))
    bounds = jnp.pad(bounds, (0, 48 - _NW - 1), constant_values=_E)
    srch = jnp.pad(src[:_N], (0, _HEAD_PAD - _N)).astype(jnp.int32)
    dsth = jnp.pad(dst[:_N], (0, _HEAD_PAD - _N)).astype(jnp.int32)

    x_proj = _mm(x, Wa, ba, True, 2000)
    r0 = _mm(edge_attr, Wb, bb, True, 2000)
    head = r0[:_N]
    h_atom = x_proj

    for l in range(3):
        if l == 0:
            aggr = sweep_linear(r0, r0, src_s, perm, bounds, node_off)[:_N]
        else:
            aggr = sweep_gather(a_tab, bt_tab, src_s, dst_s, bounds,
                                node_off)[:_N]
        h_atom, a_tab, bt_tab = _layer_tc(h_atom, aggr, head, Wseq[l], bseq[l])
        head = head_fn(a_tab, bt_tab, srch, dsth)[:_N]
    aggr = sweep_gather(a_tab, bt_tab, src_s, dst_s, bounds, node_off)[:_N]
    return _final_tc(h_atom, aggr, x_proj, Wlin, blin)
